# Initial kernel scaffold; baseline (speedup 1.0000x reference)
import jax
import jax.numpy as jnp
from jax.experimental import pallas as pl


def kernel(x, triples, num_nodes, num_relations, gene_emb, W0, b0, W1, b1, W_out, b_out):
    raise NotImplementedError("write your pallas kernel here")


# speedup vs baseline: 1.0849x; 1.0849x over previous
import functools
import math

import jax
import jax.numpy as jnp
from jax.experimental import pallas as pl

N1 = 5736
N_GENE = 4264
N_NODES = 10000
E_EDGES = 160000
R_REL = 4
HM = 128
ALPHA = 10.0
BETA = 1.0
OUTER_ITERS = 3
INNER_ITERS = 15


def _matmul_body(x_ref, w_ref, b_ref, o_ref):
    o_ref[...] = (
        jnp.dot(x_ref[...], w_ref[...], preferred_element_type=jnp.float32)
        + b_ref[...]
    )


def _tiled_matmul(x, w, b, block_m=512):
    m, k = x.shape
    n = w.shape[1]
    grid = (pl.cdiv(m, block_m),)
    return pl.pallas_call(
        _matmul_body,
        grid=grid,
        in_specs=[
            pl.BlockSpec((block_m, k), lambda i: (i, 0)),
            pl.BlockSpec((k, n), lambda i: (0, 0)),
            pl.BlockSpec((1, n), lambda i: (0, 0)),
        ],
        out_specs=pl.BlockSpec((block_m, n), lambda i: (i, 0)),
        out_shape=jax.ShapeDtypeStruct((m, n), jnp.float32),
    )(x, w, b.reshape(1, n))


def kernel(x, triples, num_nodes, num_relations, gene_emb, W0, b0, W1, b1, W_out, b_out):
    N = N_NODES
    Rt = 2 * R_REL
    x0 = x[:, 1613:]
    h0 = _tiled_matmul(x0, W0, b0)
    h1 = _tiled_matmul(gene_emb, W1, b1)
    h = jnp.concatenate([h0, h1], axis=0)

    src = triples[:, 0]
    rel = triples[:, 1]
    dst = triples[:, 2]
    src_all = jnp.concatenate([src, dst])
    rel_all = jnp.concatenate([rel, rel + R_REL])
    dst_all = jnp.concatenate([dst, src])
    rows = rel_all * N + src_all
    vals = jnp.ones(rows.shape[0], dtype=jnp.float32)
    row_sums = jnp.zeros((Rt * N,), dtype=jnp.float32).at[rows].add(vals)
    vals_norm = vals / row_sums[rows]
    deg = row_sums
    dinv = 1.0 / jnp.sqrt(jnp.maximum(deg, 1.0))

    means = jnp.mean(h, axis=1, keepdims=True)
    devs = jnp.std(h, axis=1, keepdims=True, ddof=1)
    output_ = (h - means) / devs
    output_ = jnp.where(jnp.isnan(output_), jnp.zeros_like(output_), output_)
    output = output_

    def total_variation(out):
        xs = out[src_all] * dinv[rel_all * N + src_all][:, None]
        xd = out[dst_all] * dinv[rel_all * N + dst_all][:, None]
        diff = xs - xd
        per_edge = jnp.sum(diff * diff, axis=1)
        w = jax.ops.segment_sum(per_edge, rel_all, num_segments=Rt)
        return 0.5 * w / N

    u = jnp.full((Rt,), 1.0 / Rt, dtype=jnp.float32)
    H_ = output_
    for _ in range(OUTER_ITERS):
        w = total_variation(output)
        l1tr = jnp.sum(jnp.abs(w))
        fi = l1tr + 2.0 * BETA / ALPHA
        t = 11
        for _ in range(INNER_ITERS):
            T_t = jnp.sqrt(2.0 * math.log(Rt) / (t * fi * fi))
            f_de = 2.0 * BETA / ALPHA * u + w
            u_ta = u * jnp.exp(-T_t * f_de)
            u = u_ta / jnp.sum(u_ta)
            t = t + 1
        gathered = vals_norm[:, None] * output[dst_all]
        af = jnp.zeros((Rt * N, output.shape[1]), dtype=jnp.float32).at[rows].add(gathered)
        af1 = af.reshape(Rt, N, -1)
        afw = jnp.einsum('rno,r->no', af1, u)
        output = 1.0 / (1.0 + ALPHA) * H_ + ALPHA / (1.0 + ALPHA) * afw
    embeddings = output
    out = output - jnp.mean(output, axis=0, keepdims=True)
    rms = jnp.sqrt(jnp.mean(jnp.sum(out * out, axis=1)) + 1e-6)
    out = out / rms
    logits = _tiled_matmul(out, W_out, b_out)
    return logits, embeddings


# speedup vs baseline: 6.2916x; 5.7992x over previous
import functools
import math

import jax
import jax.numpy as jnp
from jax import lax
from jax.experimental import pallas as pl
from jax.experimental.pallas import tpu as pltpu
from jax.experimental.pallas import tpu_sc as plsc

N1 = 5736
N_GENE = 4264
N_NODES = 10000
E_EDGES = 160000
R_REL = 4
HM = 128
NUM_CLASSES = 10
ALPHA = 10.0
BETA = 1.0
OUTER_ITERS = 3
INNER_ITERS = 15

NC = 2
NS = 16
NW = NC * NS
ET = 2 * E_EDGES
RT = 2 * R_REL
EPW = ET // NW
CH = 2000
NSTAGE = EPW // CH
BB = 80
NBATCH = CH // BB
NGRP = BB // 16
HIST = RT * N_NODES
HPT = HIST // NS
RPT = N_NODES // NS
ZPT = 624
ZROWS = 104
ZTAIL = N_NODES - NS * ZPT

C1 = 1.0 / (1.0 + ALPHA)
C2 = ALPHA / (1.0 + ALPHA)

_MESH = plsc.VectorSubcoreMesh(
    core_axis_name="c", subcore_axis_name="s", num_cores=NC, num_subcores=NS
)


def _wid():
    return lax.axis_index("s") * NC + lax.axis_index("c")


def _zeros16():
    return jnp.zeros((16,), jnp.float32)


def _hist_body(rel_hbm, src_hbm, hist_hbm, rel_v, src_v, idx_v, ones_v,
               zbuf_v, hist_sh):
    cid = lax.axis_index("c")
    sid = lax.axis_index("s")
    wid = _wid()

    def zfill(i, _):
        zbuf_v[pl.ds(i * 16, 16)] = _zeros16()
        return 0

    lax.fori_loop(0, 125, zfill, 0)

    def ofill(g, _):
        ones_v[pl.ds(g * 16, 16)] = jnp.ones((16,), jnp.float32)
        return 0

    lax.fori_loop(0, NGRP, ofill, 0)

    pltpu.sync_copy(zbuf_v, hist_sh.at[pl.ds(sid * HPT, 2000)])
    pltpu.sync_copy(zbuf_v, hist_sh.at[pl.ds(sid * HPT + 2000, 2000)])
    pltpu.sync_copy(zbuf_v.at[pl.ds(0, 1000)],
                    hist_sh.at[pl.ds(sid * HPT + 4000, 1000)])
    plsc.subcore_barrier()

    def stage_body(st, _):
        base = wid * EPW + st * CH
        pltpu.sync_copy(rel_hbm.at[pl.ds(base, CH)], rel_v)
        pltpu.sync_copy(src_hbm.at[pl.ds(base, CH)], src_v)

        def batch_body(b, _):
            off = b * BB
            for g in range(NGRP):
                r16 = rel_v[pl.ds(off + g * 16, 16)]
                s16 = src_v[pl.ds(off + g * 16, 16)]
                idx_v[pl.ds(g * 16, 16)] = r16 * N_NODES + s16
            pltpu.sync_copy(ones_v, hist_sh.at[idx_v], add=True)
            return 0

        lax.fori_loop(0, NBATCH, batch_body, 0)
        return 0

    lax.fori_loop(0, NSTAGE, stage_body, 0)
    plsc.subcore_barrier()
    for o, ln in ((0, 2000), (2000, 2000), (4000, 1000)):
        pltpu.sync_copy(hist_sh.at[pl.ds(sid * HPT + o, ln)],
                        zbuf_v.at[pl.ds(0, ln)])
        pltpu.sync_copy(zbuf_v.at[pl.ds(0, ln)],
                        hist_hbm.at[pl.ds(cid * HIST + sid * HPT + o, ln)])


def _sc_hist(rel_all, src_all):
    f = pl.kernel(
        _hist_body,
        out_type=jax.ShapeDtypeStruct((NC * HIST,), jnp.float32),
        mesh=_MESH,
        scratch_types=[
            pltpu.VMEM((CH,), jnp.int32),
            pltpu.VMEM((CH,), jnp.int32),
            pltpu.VMEM((BB,), jnp.int32),
            pltpu.VMEM((BB,), jnp.float32),
            pltpu.VMEM((2000,), jnp.float32),
            pltpu.VMEM_SHARED((HIST,), jnp.float32),
        ],
    )
    return f(rel_all, src_all)


def _tv_body(out_hbm, dinv_hbm, rel_hbm, src_hbm, dst_hbm, wraw_hbm,
             rel_v, src_v, dst_v, sbuf, dbuf, sidx_v, gidx_v,
             rowidx_v, colidx_v, dsb_v, ddb_v,
             wv8, sem0, sem1, sem2, sem3):
    wid = _wid()

    def stage_body(st, wacc):
        base = wid * EPW + st * CH
        pltpu.sync_copy(rel_hbm.at[pl.ds(base, CH)], rel_v)
        pltpu.sync_copy(src_hbm.at[pl.ds(base, CH)], src_v)
        pltpu.sync_copy(dst_hbm.at[pl.ds(base, CH)], dst_v)

        def batch_body(b, wacc):
            off = b * BB
            for g in range(NGRP):
                r16 = rel_v[pl.ds(off + g * 16, 16)]
                s16 = src_v[pl.ds(off + g * 16, 16)]
                d16 = dst_v[pl.ds(off + g * 16, 16)]
                sidx_v[pl.ds(g * 16, 16)] = s16
                gidx_v[pl.ds(g * 16, 16)] = d16
                rowidx_v[pl.ds(g * 16, 16)] = r16 * N_NODES + s16
                colidx_v[pl.ds(g * 16, 16)] = r16 * N_NODES + d16
            cp0 = pltpu.async_copy(out_hbm.at[sidx_v], sbuf, sem0)
            cp1 = pltpu.async_copy(out_hbm.at[gidx_v], dbuf, sem1)
            cp2 = pltpu.async_copy(dinv_hbm.at[rowidx_v], dsb_v, sem2)
            cp3 = pltpu.async_copy(dinv_hbm.at[colidx_v], ddb_v, sem3)
            cp0.wait()
            cp1.wait()
            cp2.wait()
            cp3.wait()

            for g in range(NGRP):
                ds16 = dsb_v[pl.ds(g * 16, 16)]
                dd16 = ddb_v[pl.ds(g * 16, 16)]
                r16 = rel_v[pl.ds(off + g * 16, 16)]
                for ei in range(16):
                    e = g * 16 + ei
                    dse = ds16[ei]
                    dde = dd16[ei]
                    re_ = r16[ei]
                    pacc = _zeros16()
                    for k in range(HM // 16):
                        t = (dse * sbuf[e, pl.ds(k * 16, 16)]
                             - dde * dbuf[e, pl.ds(k * 16, 16)])
                        pacc = pacc + t * t
                    wacc = tuple(
                        jnp.where(re_ == r, wacc[r] + pacc, wacc[r])
                        for r in range(RT)
                    )
            return wacc

        return lax.fori_loop(0, NBATCH, batch_body, wacc)

    wacc = lax.fori_loop(0, NSTAGE, stage_body, (_zeros16(),) * RT)
    for r in range(RT):
        wv8[pl.ds(r * 16, 16)] = wacc[r]
    pltpu.sync_copy(wv8, wraw_hbm.at[pl.ds(wid * RT * 16, RT * 16)])


def _sc_tv(out, dinv, rel_all, src_all, dst_all):
    f = pl.kernel(
        _tv_body,
        out_type=jax.ShapeDtypeStruct((NW * RT * 16,), jnp.float32),
        mesh=_MESH,
        scratch_types=[
            pltpu.VMEM((CH,), jnp.int32),
            pltpu.VMEM((CH,), jnp.int32),
            pltpu.VMEM((CH,), jnp.int32),
            pltpu.VMEM((BB, HM), jnp.float32),
            pltpu.VMEM((BB, HM), jnp.float32),
            pltpu.VMEM((BB,), jnp.int32),
            pltpu.VMEM((BB,), jnp.int32),
            pltpu.VMEM((BB,), jnp.int32),
            pltpu.VMEM((BB,), jnp.int32),
            pltpu.VMEM((BB,), jnp.float32),
            pltpu.VMEM((BB,), jnp.float32),
            pltpu.VMEM((RT * 16,), jnp.float32),
            pltpu.SemaphoreType.DMA,
            pltpu.SemaphoreType.DMA,
            pltpu.SemaphoreType.DMA,
            pltpu.SemaphoreType.DMA,
        ],
    )
    return f(out, dinv, rel_all, src_all, dst_all)


def _prop_body(out_hbm, dinv_hbm, rel_hbm, src_hbm, dst_hbm, u_hbm,
               afw_hbm, rel_v, src_v, dst_v, dbuf, zbuf, sidx_v,
               gidx_v, rowidx_v, dsb_v, u_v, afw_sh, sem0, sem1):
    cid = lax.axis_index("c")
    sid = lax.axis_index("s")
    wid = _wid()
    pltpu.sync_copy(u_hbm, u_v)
    uvec = u_v[...]
    u_sc = [uvec[r] for r in range(RT)]

    def zfill(i, _):
        for k in range(HM // 16):
            zbuf[i, pl.ds(k * 16, 16)] = _zeros16()
        return 0

    lax.fori_loop(0, ZROWS, zfill, 0)
    for k in range(ZPT // ZROWS):
        pltpu.sync_copy(zbuf, afw_sh.at[pl.ds(sid * ZPT + k * ZROWS, ZROWS)])

    @pl.when(sid == 0)
    def _():
        pltpu.sync_copy(zbuf.at[pl.ds(0, ZTAIL)],
                        afw_sh.at[pl.ds(NS * ZPT, ZTAIL)])

    plsc.subcore_barrier()

    def stage_body(st, _):
        base = wid * EPW + st * CH
        pltpu.sync_copy(rel_hbm.at[pl.ds(base, CH)], rel_v)
        pltpu.sync_copy(src_hbm.at[pl.ds(base, CH)], src_v)
        pltpu.sync_copy(dst_hbm.at[pl.ds(base, CH)], dst_v)

        def batch_body(b, _):
            off = b * BB
            for g in range(NGRP):
                r16 = rel_v[pl.ds(off + g * 16, 16)]
                s16 = src_v[pl.ds(off + g * 16, 16)]
                d16 = dst_v[pl.ds(off + g * 16, 16)]
                sidx_v[pl.ds(g * 16, 16)] = s16
                gidx_v[pl.ds(g * 16, 16)] = d16
                rowidx_v[pl.ds(g * 16, 16)] = r16 * N_NODES + s16
            cp0 = pltpu.async_copy(out_hbm.at[gidx_v], dbuf, sem0)
            cp1 = pltpu.async_copy(dinv_hbm.at[rowidx_v], dsb_v, sem1)
            cp0.wait()
            cp1.wait()

            for g in range(NGRP):
                ds16 = dsb_v[pl.ds(g * 16, 16)]
                r16 = rel_v[pl.ds(off + g * 16, 16)]
                ug = _zeros16()
                for r in range(RT):
                    ug = jnp.where(r16 == r, u_sc[r], ug)
                we16 = ug * ds16 * ds16
                for ei in range(16):
                    e = g * 16 + ei
                    we = we16[ei]
                    for k in range(HM // 16):
                        sl = pl.ds(k * 16, 16)
                        dbuf[e, sl] = dbuf[e, sl] * we
            pltpu.sync_copy(dbuf, afw_sh.at[sidx_v], add=True)
            return 0

        lax.fori_loop(0, NBATCH, batch_body, 0)
        return 0

    lax.fori_loop(0, NSTAGE, stage_body, 0)
    plsc.subcore_barrier()
    for k in range(ZPT // ZROWS):
        b = sid * ZPT + k * ZROWS
        pltpu.sync_copy(afw_sh.at[pl.ds(b, ZROWS)], zbuf)
        pltpu.sync_copy(zbuf, afw_hbm.at[cid, pl.ds(b, ZROWS)])

    @pl.when(sid == 0)
    def _():
        pltpu.sync_copy(afw_sh.at[pl.ds(NS * ZPT, ZTAIL)],
                        zbuf.at[pl.ds(0, ZTAIL)])
        pltpu.sync_copy(zbuf.at[pl.ds(0, ZTAIL)],
                        afw_hbm.at[cid, pl.ds(NS * ZPT, ZTAIL)])


def _sc_prop(out, dinv, rel_all, src_all, dst_all, u16):
    f = pl.kernel(
        _prop_body,
        out_type=jax.ShapeDtypeStruct((NC, N_NODES, HM), jnp.float32),
        mesh=_MESH,
        scratch_types=[
            pltpu.VMEM((CH,), jnp.int32),
            pltpu.VMEM((CH,), jnp.int32),
            pltpu.VMEM((CH,), jnp.int32),
            pltpu.VMEM((BB, HM), jnp.float32),
            pltpu.VMEM((ZROWS, HM), jnp.float32),
            pltpu.VMEM((BB,), jnp.int32),
            pltpu.VMEM((BB,), jnp.int32),
            pltpu.VMEM((BB,), jnp.int32),
            pltpu.VMEM((BB,), jnp.float32),
            pltpu.VMEM((16,), jnp.float32),
            pltpu.VMEM_SHARED((N_NODES, HM), jnp.float32),
            pltpu.SemaphoreType.DMA,
            pltpu.SemaphoreType.DMA,
        ],
    )
    return f(out, dinv, rel_all, src_all, dst_all, u16)


def _xform_body(x_ref, w_ref, b_ref, o_ref):
    h = (
        jnp.dot(x_ref[...], w_ref[...], preferred_element_type=jnp.float32)
        + b_ref[...]
    )
    m = jnp.mean(h, axis=1, keepdims=True)
    d = h - m
    var = jnp.sum(d * d, axis=1, keepdims=True) * (1.0 / (HM - 1))
    t = d / jnp.sqrt(var)
    o_ref[...] = jnp.where(jnp.isnan(t), 0.0, t)


def _tc_xform(x, w, b, block_m=512):
    m, k = x.shape
    n = w.shape[1]
    return pl.pallas_call(
        _xform_body,
        grid=(pl.cdiv(m, block_m),),
        in_specs=[
            pl.BlockSpec((block_m, k), lambda i: (i, 0)),
            pl.BlockSpec((k, n), lambda i: (0, 0)),
            pl.BlockSpec((1, n), lambda i: (0, 0)),
        ],
        out_specs=pl.BlockSpec((block_m, n), lambda i: (i, 0)),
        out_shape=jax.ShapeDtypeStruct((m, n), jnp.float32),
    )(x, w, b.reshape(1, n))


def _dinv_body(h_ref, o_ref):
    deg = h_ref[0] + h_ref[1]
    o_ref[...] = lax.rsqrt(jnp.maximum(deg, 1.0))


def _tc_dinv(hist2flat):
    h3 = hist2flat.reshape(NC, HIST // HM, HM)
    out = pl.pallas_call(
        _dinv_body,
        out_shape=jax.ShapeDtypeStruct((HIST // HM, HM), jnp.float32),
    )(h3)
    return out.reshape(HIST)


def _uupd_body(w_ref, o_ref):
    w = jnp.sum(w_ref[...], axis=(0, 2)) * (0.5 / float(N_NODES))
    l1tr = jnp.sum(jnp.abs(w))
    fi = l1tr + 2.0 * BETA / ALPHA
    u = jnp.full((RT,), 1.0 / RT, jnp.float32)
    for t in range(11, 11 + INNER_ITERS):
        tt = math.sqrt(2.0 * math.log(float(RT)) / float(t)) / fi
        f_de = (2.0 * BETA / ALPHA) * u + w
        u_ta = u * jnp.exp(-tt * f_de)
        u = u_ta / jnp.sum(u_ta)
    o_ref[...] = jnp.concatenate(
        [u, jnp.zeros((RT,), jnp.float32)]
    ).reshape(1, 16)


def _tc_uupd(wraw):
    return pl.pallas_call(
        _uupd_body,
        out_shape=jax.ShapeDtypeStruct((1, 16), jnp.float32),
    )(wraw.reshape(NW, RT, 16))


def _update_body(h_ref, a_ref, b_ref, o_ref):
    o_ref[...] = C1 * h_ref[...] + C2 * (a_ref[...] + b_ref[...])


def _tc_update(h0, afw0, afw1, block_m=512):
    m, n = h0.shape
    spec = pl.BlockSpec((block_m, n), lambda i: (i, 0))
    return pl.pallas_call(
        _update_body,
        grid=(pl.cdiv(m, block_m),),
        in_specs=[spec, spec, spec],
        out_specs=spec,
        out_shape=jax.ShapeDtypeStruct((m, n), jnp.float32),
    )(h0, afw0, afw1)


def _epi_body(x_ref, w_ref, b_ref, o_ref):
    x = x_ref[...]
    xc = x - jnp.mean(x, axis=0, keepdims=True)
    rms = jnp.sqrt(jnp.mean(jnp.sum(xc * xc, axis=1)) + 1e-6)
    xn = xc / rms
    o_ref[...] = (
        jnp.dot(xn, w_ref[...], preferred_element_type=jnp.float32)
        + b_ref[...]
    )


def _tc_epilogue(x, w_out, b_out):
    m, n = x.shape
    c = w_out.shape[1]
    return pl.pallas_call(
        _epi_body,
        out_shape=jax.ShapeDtypeStruct((m, c), jnp.float32),
    )(x, w_out, b_out.reshape(1, c))


def kernel(x, triples, num_nodes, num_relations, gene_emb, W0, b0, W1, b1,
           W_out, b_out):
    src = triples[:, 0].astype(jnp.int32)
    rel = triples[:, 1].astype(jnp.int32)
    dst = triples[:, 2].astype(jnp.int32)
    src_all = jnp.concatenate([src, dst])
    rel_all = jnp.concatenate([rel, rel + R_REL])
    dst_all = jnp.concatenate([dst, src])

    h0 = _tc_xform(x[:, 1613:], W0, b0)
    h1 = _tc_xform(gene_emb, W1, b1)
    output0 = jnp.concatenate([h0, h1], axis=0)

    hist2 = _sc_hist(rel_all, src_all)
    dinv = _tc_dinv(hist2)

    output = output0
    for _ in range(OUTER_ITERS):
        wraw = _sc_tv(output, dinv, rel_all, src_all, dst_all)
        u16 = _tc_uupd(wraw).reshape(16)
        afw = _sc_prop(output, dinv, rel_all, src_all, dst_all, u16)
        output = _tc_update(output0, afw[0], afw[1])

    logits = _tc_epilogue(output, W_out, b_out)
    return logits, output


# speedup vs baseline: 7.9964x; 1.2710x over previous
import functools
import math

import jax
import jax.numpy as jnp
from jax import lax
from jax.experimental import pallas as pl
from jax.experimental.pallas import tpu as pltpu
from jax.experimental.pallas import tpu_sc as plsc

N1 = 5736
N_GENE = 4264
N_NODES = 10000
E_EDGES = 160000
R_REL = 4
HM = 128
NUM_CLASSES = 10
ALPHA = 10.0
BETA = 1.0
OUTER_ITERS = 3
INNER_ITERS = 15

NC = 2
NS = 16
NW = NC * NS
ET = 2 * E_EDGES
RT = 2 * R_REL
EPW = ET // NW
CH = 2000
NSTAGE = EPW // CH
BB = 80
NBATCH = CH // BB
NGRP = BB // 16
HIST = RT * N_NODES
HPT = HIST // NS
RPT = N_NODES // NS
ZPT = 624
ZROWS = 104
ZTAIL = N_NODES - NS * ZPT

C1 = 1.0 / (1.0 + ALPHA)
C2 = ALPHA / (1.0 + ALPHA)

_MESH = plsc.VectorSubcoreMesh(
    core_axis_name="c", subcore_axis_name="s", num_cores=NC, num_subcores=NS
)


def _wid():
    return lax.axis_index("s") * NC + lax.axis_index("c")


def _zeros16():
    return jnp.zeros((16,), jnp.float32)


def _hist_body(rel_hbm, src_hbm, hist_hbm, rel_v, src_v, idx_v, ones_v,
               zbuf_v, hist_sh):
    cid = lax.axis_index("c")
    sid = lax.axis_index("s")
    wid = _wid()

    def zfill(i, _):
        zbuf_v[pl.ds(i * 16, 16)] = _zeros16()
        return 0

    lax.fori_loop(0, 125, zfill, 0)

    def ofill(g, _):
        ones_v[pl.ds(g * 16, 16)] = jnp.ones((16,), jnp.float32)
        return 0

    lax.fori_loop(0, NGRP, ofill, 0)

    pltpu.sync_copy(zbuf_v, hist_sh.at[pl.ds(sid * HPT, 2000)])
    pltpu.sync_copy(zbuf_v, hist_sh.at[pl.ds(sid * HPT + 2000, 2000)])
    pltpu.sync_copy(zbuf_v.at[pl.ds(0, 1000)],
                    hist_sh.at[pl.ds(sid * HPT + 4000, 1000)])
    plsc.subcore_barrier()

    def stage_body(st, _):
        base = wid * EPW + st * CH
        pltpu.sync_copy(rel_hbm.at[pl.ds(base, CH)], rel_v)
        pltpu.sync_copy(src_hbm.at[pl.ds(base, CH)], src_v)

        def batch_body(b, _):
            off = b * BB
            for g in range(NGRP):
                r16 = rel_v[pl.ds(off + g * 16, 16)]
                s16 = src_v[pl.ds(off + g * 16, 16)]
                idx_v[pl.ds(g * 16, 16)] = r16 * N_NODES + s16
            pltpu.sync_copy(ones_v, hist_sh.at[idx_v], add=True)
            return 0

        lax.fori_loop(0, NBATCH, batch_body, 0)
        return 0

    lax.fori_loop(0, NSTAGE, stage_body, 0)
    plsc.subcore_barrier()
    for o, ln in ((0, 2000), (2000, 2000), (4000, 1000)):
        pltpu.sync_copy(hist_sh.at[pl.ds(sid * HPT + o, ln)],
                        zbuf_v.at[pl.ds(0, ln)])
        pltpu.sync_copy(zbuf_v.at[pl.ds(0, ln)],
                        hist_hbm.at[pl.ds(cid * HIST + sid * HPT + o, ln)])


def _sc_hist(rel_all, src_all):
    f = pl.kernel(
        _hist_body,
        out_type=jax.ShapeDtypeStruct((NC * HIST,), jnp.float32),
        mesh=_MESH,
        scratch_types=[
            pltpu.VMEM((CH,), jnp.int32),
            pltpu.VMEM((CH,), jnp.int32),
            pltpu.VMEM((BB,), jnp.int32),
            pltpu.VMEM((BB,), jnp.float32),
            pltpu.VMEM((2000,), jnp.float32),
            pltpu.VMEM_SHARED((HIST,), jnp.float32),
        ],
    )
    return f(rel_all, src_all)


def _tv_body(out_hbm, dinv_hbm, rel_hbm, src_hbm, dst_hbm, wraw_hbm,
             rel_v, src_v, dst_v, sbufA, sbufB, dbufA, dbufB,
             sidxA, sidxB, gidxA, gidxB, rowA, rowB, colA, colB,
             dsbA, dsbB, ddbA, ddbB, wv8, semA, semB):
    wid = _wid()
    sbuf = (sbufA, sbufB)
    dbuf = (dbufA, dbufB)
    sidx = (sidxA, sidxB)
    gidx = (gidxA, gidxB)
    row = (rowA, rowB)
    col = (colA, colB)
    dsb = (dsbA, dsbB)
    ddb = (ddbA, ddbB)
    sem = (semA, semB)

    def _copies(p):
        return (
            pltpu.make_async_copy(out_hbm.at[sidx[p]], sbuf[p], sem[p]),
            pltpu.make_async_copy(out_hbm.at[gidx[p]], dbuf[p], sem[p]),
            pltpu.make_async_copy(dinv_hbm.at[row[p]], dsb[p], sem[p]),
            pltpu.make_async_copy(dinv_hbm.at[col[p]], ddb[p], sem[p]),
        )

    def _fire(b, p):
        off = b * BB
        for g in range(NGRP):
            r16 = rel_v[pl.ds(off + g * 16, 16)]
            s16 = src_v[pl.ds(off + g * 16, 16)]
            d16 = dst_v[pl.ds(off + g * 16, 16)]
            sidx[p][pl.ds(g * 16, 16)] = s16
            gidx[p][pl.ds(g * 16, 16)] = d16
            row[p][pl.ds(g * 16, 16)] = r16 * N_NODES + s16
            col[p][pl.ds(g * 16, 16)] = r16 * N_NODES + d16
        for cp in _copies(p):
            cp.start()

    def _drain(p):
        for cp in _copies(p):
            cp.wait()

    def _compute(b, p):
        off = b * BB
        wacc = tuple(wv8[pl.ds(r * 16, 16)] for r in range(RT))
        for g in range(NGRP):
            ds16 = dsb[p][pl.ds(g * 16, 16)]
            dd16 = ddb[p][pl.ds(g * 16, 16)]
            r16 = rel_v[pl.ds(off + g * 16, 16)]
            for ei in range(16):
                e = g * 16 + ei
                dse = ds16[ei]
                dde = dd16[ei]
                re_ = r16[ei]
                pacc = _zeros16()
                for k in range(HM // 16):
                    t = (dse * sbuf[p][e, pl.ds(k * 16, 16)]
                         - dde * dbuf[p][e, pl.ds(k * 16, 16)])
                    pacc = pacc + t * t
                wacc = tuple(
                    jnp.where(re_ == r, wacc[r] + pacc, wacc[r])
                    for r in range(RT)
                )
        for r in range(RT):
            wv8[pl.ds(r * 16, 16)] = wacc[r]

    for r in range(RT):
        wv8[pl.ds(r * 16, 16)] = _zeros16()

    def stage_body(st, _):
        base = wid * EPW + st * CH
        pltpu.sync_copy(rel_hbm.at[pl.ds(base, CH)], rel_v)
        pltpu.sync_copy(src_hbm.at[pl.ds(base, CH)], src_v)
        pltpu.sync_copy(dst_hbm.at[pl.ds(base, CH)], dst_v)
        _fire(0, 0)

        def pair_body(i, _):
            b0 = 2 * i
            b1 = b0 + 1

            @pl.when(b1 < NBATCH)
            def _():
                _fire(b1, 1)

            _drain(0)
            _compute(b0, 0)

            @pl.when(b0 + 2 < NBATCH)
            def _():
                _fire(b0 + 2, 0)

            @pl.when(b1 < NBATCH)
            def _():
                _drain(1)
                _compute(b1, 1)

            return 0

        lax.fori_loop(0, (NBATCH + 1) // 2, pair_body, 0)
        return 0

    lax.fori_loop(0, NSTAGE, stage_body, 0)
    pltpu.sync_copy(wv8, wraw_hbm.at[pl.ds(wid * RT * 16, RT * 16)])


def _sc_tv(out, dinv, rel_all, src_all, dst_all):
    f = pl.kernel(
        _tv_body,
        out_type=jax.ShapeDtypeStruct((NW * RT * 16,), jnp.float32),
        mesh=_MESH,
        scratch_types=(
            [pltpu.VMEM((CH,), jnp.int32)] * 3
            + [pltpu.VMEM((BB, HM), jnp.float32)] * 4
            + [pltpu.VMEM((BB,), jnp.int32)] * 8
            + [pltpu.VMEM((BB,), jnp.float32)] * 4
            + [pltpu.VMEM((RT * 16,), jnp.float32)]
            + [pltpu.SemaphoreType.DMA] * 2
        ),
    )
    return f(out, dinv, rel_all, src_all, dst_all)


def _prop_body(out_hbm, dinv_hbm, rel_hbm, src_hbm, dst_hbm, u_hbm,
               afw_hbm, rel_v, src_v, dst_v, dbufA, dbufB, zbuf,
               sidxA, sidxB, gidxA, gidxB, rowA, rowB, dsbA, dsbB,
               u_v, afw_sh, semA, semB):
    cid = lax.axis_index("c")
    sid = lax.axis_index("s")
    wid = _wid()
    dbuf = (dbufA, dbufB)
    sidx = (sidxA, sidxB)
    gidx = (gidxA, gidxB)
    row = (rowA, rowB)
    dsb = (dsbA, dsbB)
    sem = (semA, semB)
    pltpu.sync_copy(u_hbm, u_v)
    uvec = u_v[...]
    u_sc = [uvec[r] for r in range(RT)]

    def zfill(i, _):
        for k in range(HM // 16):
            zbuf[i, pl.ds(k * 16, 16)] = _zeros16()
        return 0

    lax.fori_loop(0, ZROWS, zfill, 0)
    for k in range(ZPT // ZROWS):
        pltpu.sync_copy(zbuf, afw_sh.at[pl.ds(sid * ZPT + k * ZROWS, ZROWS)])

    @pl.when(sid == 0)
    def _():
        pltpu.sync_copy(zbuf.at[pl.ds(0, ZTAIL)],
                        afw_sh.at[pl.ds(NS * ZPT, ZTAIL)])

    plsc.subcore_barrier()

    def _copies(p):
        return (
            pltpu.make_async_copy(out_hbm.at[gidx[p]], dbuf[p], sem[p]),
            pltpu.make_async_copy(dinv_hbm.at[row[p]], dsb[p], sem[p]),
        )

    def _fire(b, p):
        off = b * BB
        for g in range(NGRP):
            r16 = rel_v[pl.ds(off + g * 16, 16)]
            s16 = src_v[pl.ds(off + g * 16, 16)]
            d16 = dst_v[pl.ds(off + g * 16, 16)]
            sidx[p][pl.ds(g * 16, 16)] = s16
            gidx[p][pl.ds(g * 16, 16)] = d16
            row[p][pl.ds(g * 16, 16)] = r16 * N_NODES + s16
        for cp in _copies(p):
            cp.start()

    def _drain(p):
        for cp in _copies(p):
            cp.wait()

    def _compute(b, p):
        off = b * BB
        for g in range(NGRP):
            ds16 = dsb[p][pl.ds(g * 16, 16)]
            r16 = rel_v[pl.ds(off + g * 16, 16)]
            ug = _zeros16()
            for r in range(RT):
                ug = jnp.where(r16 == r, u_sc[r], ug)
            we16 = ug * ds16 * ds16
            for ei in range(16):
                e = g * 16 + ei
                we = we16[ei]
                for k in range(HM // 16):
                    sl = pl.ds(k * 16, 16)
                    dbuf[p][e, sl] = dbuf[p][e, sl] * we
        pltpu.sync_copy(dbuf[p], afw_sh.at[sidx[p]], add=True)

    def stage_body(st, _):
        base = wid * EPW + st * CH
        pltpu.sync_copy(rel_hbm.at[pl.ds(base, CH)], rel_v)
        pltpu.sync_copy(src_hbm.at[pl.ds(base, CH)], src_v)
        pltpu.sync_copy(dst_hbm.at[pl.ds(base, CH)], dst_v)
        _fire(0, 0)

        def pair_body(i, _):
            b0 = 2 * i
            b1 = b0 + 1

            @pl.when(b1 < NBATCH)
            def _():
                _fire(b1, 1)

            _drain(0)
            _compute(b0, 0)

            @pl.when(b0 + 2 < NBATCH)
            def _():
                _fire(b0 + 2, 0)

            @pl.when(b1 < NBATCH)
            def _():
                _drain(1)
                _compute(b1, 1)

            return 0

        lax.fori_loop(0, (NBATCH + 1) // 2, pair_body, 0)
        return 0

    lax.fori_loop(0, NSTAGE, stage_body, 0)
    plsc.subcore_barrier()
    for k in range(ZPT // ZROWS):
        b = sid * ZPT + k * ZROWS
        pltpu.sync_copy(afw_sh.at[pl.ds(b, ZROWS)], zbuf)
        pltpu.sync_copy(zbuf, afw_hbm.at[cid, pl.ds(b, ZROWS)])

    @pl.when(sid == 0)
    def _():
        pltpu.sync_copy(afw_sh.at[pl.ds(NS * ZPT, ZTAIL)],
                        zbuf.at[pl.ds(0, ZTAIL)])
        pltpu.sync_copy(zbuf.at[pl.ds(0, ZTAIL)],
                        afw_hbm.at[cid, pl.ds(NS * ZPT, ZTAIL)])


def _sc_prop(out, dinv, rel_all, src_all, dst_all, u16):
    f = pl.kernel(
        _prop_body,
        out_type=jax.ShapeDtypeStruct((NC, N_NODES, HM), jnp.float32),
        mesh=_MESH,
        scratch_types=(
            [pltpu.VMEM((CH,), jnp.int32)] * 3
            + [pltpu.VMEM((BB, HM), jnp.float32)] * 2
            + [pltpu.VMEM((ZROWS, HM), jnp.float32)]
            + [pltpu.VMEM((BB,), jnp.int32)] * 6
            + [pltpu.VMEM((BB,), jnp.float32)] * 2
            + [pltpu.VMEM((16,), jnp.float32)]
            + [pltpu.VMEM_SHARED((N_NODES, HM), jnp.float32)]
            + [pltpu.SemaphoreType.DMA] * 2
        ),
    )
    return f(out, dinv, rel_all, src_all, dst_all, u16)


def _xform_body(x_ref, w_ref, b_ref, o_ref):
    h = (
        jnp.dot(x_ref[...], w_ref[...], preferred_element_type=jnp.float32)
        + b_ref[...]
    )
    m = jnp.mean(h, axis=1, keepdims=True)
    d = h - m
    var = jnp.sum(d * d, axis=1, keepdims=True) * (1.0 / (HM - 1))
    t = d / jnp.sqrt(var)
    o_ref[...] = jnp.where(jnp.isnan(t), 0.0, t)


def _tc_xform(x, w, b, block_m=512):
    m, k = x.shape
    n = w.shape[1]
    return pl.pallas_call(
        _xform_body,
        grid=(pl.cdiv(m, block_m),),
        in_specs=[
            pl.BlockSpec((block_m, k), lambda i: (i, 0)),
            pl.BlockSpec((k, n), lambda i: (0, 0)),
            pl.BlockSpec((1, n), lambda i: (0, 0)),
        ],
        out_specs=pl.BlockSpec((block_m, n), lambda i: (i, 0)),
        out_shape=jax.ShapeDtypeStruct((m, n), jnp.float32),
    )(x, w, b.reshape(1, n))


def _dinv_body(h_ref, o_ref):
    deg = h_ref[0] + h_ref[1]
    o_ref[...] = lax.rsqrt(jnp.maximum(deg, 1.0))


def _tc_dinv(hist2flat):
    h3 = hist2flat.reshape(NC, HIST // HM, HM)
    out = pl.pallas_call(
        _dinv_body,
        out_shape=jax.ShapeDtypeStruct((HIST // HM, HM), jnp.float32),
    )(h3)
    return out.reshape(HIST)


def _uupd_body(w_ref, o_ref):
    w = jnp.sum(w_ref[...], axis=(0, 2)) * (0.5 / float(N_NODES))
    l1tr = jnp.sum(jnp.abs(w))
    fi = l1tr + 2.0 * BETA / ALPHA
    u = jnp.full((RT,), 1.0 / RT, jnp.float32)
    for t in range(11, 11 + INNER_ITERS):
        tt = math.sqrt(2.0 * math.log(float(RT)) / float(t)) / fi
        f_de = (2.0 * BETA / ALPHA) * u + w
        u_ta = u * jnp.exp(-tt * f_de)
        u = u_ta / jnp.sum(u_ta)
    o_ref[...] = jnp.concatenate(
        [u, jnp.zeros((RT,), jnp.float32)]
    ).reshape(1, 16)


def _tc_uupd(wraw):
    return pl.pallas_call(
        _uupd_body,
        out_shape=jax.ShapeDtypeStruct((1, 16), jnp.float32),
    )(wraw.reshape(NW, RT, 16))


def _update_body(h_ref, a_ref, b_ref, o_ref):
    o_ref[...] = C1 * h_ref[...] + C2 * (a_ref[...] + b_ref[...])


def _tc_update(h0, afw0, afw1, block_m=512):
    m, n = h0.shape
    spec = pl.BlockSpec((block_m, n), lambda i: (i, 0))
    return pl.pallas_call(
        _update_body,
        grid=(pl.cdiv(m, block_m),),
        in_specs=[spec, spec, spec],
        out_specs=spec,
        out_shape=jax.ShapeDtypeStruct((m, n), jnp.float32),
    )(h0, afw0, afw1)


def _epi_body(x_ref, w_ref, b_ref, o_ref):
    x = x_ref[...]
    xc = x - jnp.mean(x, axis=0, keepdims=True)
    rms = jnp.sqrt(jnp.mean(jnp.sum(xc * xc, axis=1)) + 1e-6)
    xn = xc / rms
    o_ref[...] = (
        jnp.dot(xn, w_ref[...], preferred_element_type=jnp.float32)
        + b_ref[...]
    )


def _tc_epilogue(x, w_out, b_out):
    m, n = x.shape
    c = w_out.shape[1]
    return pl.pallas_call(
        _epi_body,
        out_shape=jax.ShapeDtypeStruct((m, c), jnp.float32),
    )(x, w_out, b_out.reshape(1, c))


def kernel(x, triples, num_nodes, num_relations, gene_emb, W0, b0, W1, b1,
           W_out, b_out):
    src = triples[:, 0].astype(jnp.int32)
    rel = triples[:, 1].astype(jnp.int32)
    dst = triples[:, 2].astype(jnp.int32)
    src_all = jnp.concatenate([src, dst])
    rel_all = jnp.concatenate([rel, rel + R_REL])
    dst_all = jnp.concatenate([dst, src])

    h0 = _tc_xform(x[:, 1613:], W0, b0)
    h1 = _tc_xform(gene_emb, W1, b1)
    output0 = jnp.concatenate([h0, h1], axis=0)

    hist2 = _sc_hist(rel_all, src_all)
    dinv = _tc_dinv(hist2)

    output = output0
    for _ in range(OUTER_ITERS):
        wraw = _sc_tv(output, dinv, rel_all, src_all, dst_all)
        u16 = _tc_uupd(wraw).reshape(16)
        afw = _sc_prop(output, dinv, rel_all, src_all, dst_all, u16)
        output = _tc_update(output0, afw[0], afw[1])

    logits = _tc_epilogue(output, W_out, b_out)
    return logits, output
